# Initial kernel scaffold; baseline (speedup 1.0000x reference)
import jax
import jax.numpy as jnp
from jax.experimental import pallas as pl


def kernel(z_ticker, z_institution, z_mutual_fund, z_news, ei_holds_it, ei_holds_mt, ei_about_nt, ei_rev_holds_mt, ei_rev_holds_it, ei_rev_about_nt, Wl_hi, bl_hi, Wr_hi, Wl_hm, bl_hm, Wr_hm, Wl_an, bl_an, Wr_an, Wl_rhm, bl_rhm, Wr_rhm, Wl_rhi, bl_rhi, Wr_rhi, Wl_ran, bl_ran, Wr_ran):
    raise NotImplementedError("write your pallas kernel here")


# speedup vs baseline: 3.6574x; 3.6574x over previous
import functools

import jax
import jax.numpy as jnp
from jax import lax
from jax.experimental import pallas as pl
from jax.experimental.pallas import tpu as pltpu
from jax.experimental.pallas import tpu_sc as plsc

N = 10000
D = 128
E = 320000
CNTW = 16
NS = 16
EDGES_PER_TILE = E // NS
CH = 80
NCHUNK = EDGES_PER_TILE // CH
STRIPE = 624
TAIL0 = STRIPE * NS
TAILR = N - TAIL0

_REL_TABLE = (0, 0, 3, 2, 1, 0)
_CORE_RELS = ((0, 1, 2), (3, 4, 5))


def _sc_segment_sums(tabs, srcs, dsts, zf, zc, ones_h):
    mesh = plsc.VectorSubcoreMesh(core_axis_name="c", subcore_axis_name="s",
                                  num_cores=2, num_subcores=NS)
    out_type = tuple(
        [jax.ShapeDtypeStruct((N, D), jnp.float32) for _ in range(6)]
        + [jax.ShapeDtypeStruct((N, CNTW), jnp.float32) for _ in range(6)]
    )

    @functools.partial(
        pl.kernel,
        mesh=mesh,
        out_type=out_type,
        compiler_params=pltpu.CompilerParams(use_tc_tiling_on_sc=False,
                                             has_side_effects=True),
        scratch_types=[
            pltpu.VMEM_SHARED((N, D), jnp.float32),
            pltpu.VMEM_SHARED((N, CNTW), jnp.float32),
            pltpu.VMEM((CH,), jnp.int32),
            pltpu.VMEM((CH,), jnp.int32),
            pltpu.VMEM((CH, D), jnp.float32),
            pltpu.VMEM((CH, CNTW), jnp.float32),
            pltpu.VMEM((CH, CNTW), jnp.float32),
            pltpu.SemaphoreType.DMA,
        ],
    )
    def k(tab0, tab1, tab2, tab3,
          s0, s1, s2, s3, s4, s5, d0, d1, d2, d3, d4, d5,
          zf_h, zc_h, ones_hbm,
          a0, a1, a2, a3, a4, a5, c0, c1, c2, c3, c4, c5,
          accf, accc, sidx, didx, rows, ones_v, cbuf, sem):
        cid = lax.axis_index("c")
        sid = lax.axis_index("s")
        tables = (tab0, tab1, tab2, tab3)
        srcs = (s0, s1, s2, s3, s4, s5)
        dsts = (d0, d1, d2, d3, d4, d5)
        aggs = (a0, a1, a2, a3, a4, a5)
        cnts = (c0, c1, c2, c3, c4, c5)
        pltpu.sync_copy(ones_hbm, ones_v)
        row0 = pl.multiple_of(sid * STRIPE, 8)
        ebase = pl.multiple_of(sid * EDGES_PER_TILE, 8)

        def stripe_chunks():
            return [(pl.multiple_of(row0 + o, 8), min(CH, STRIPE - o))
                    for o in range(0, STRIPE, CH)]

        def stripe_out(buf, dst_ref):
            for o, nr in stripe_chunks():
                pltpu.sync_copy(buf.at[pl.ds(0, nr)], dst_ref.at[pl.ds(o, nr)])

            @pl.when(sid == NS - 1)
            def _():
                pltpu.sync_copy(buf.at[pl.ds(0, TAILR)],
                                dst_ref.at[pl.ds(TAIL0, TAILR)])

        def stripe_flush(acc_ref, buf, out_ref):
            for o, nr in stripe_chunks():
                pltpu.sync_copy(acc_ref.at[pl.ds(o, nr)], buf.at[pl.ds(0, nr)])
                pltpu.sync_copy(buf.at[pl.ds(0, nr)], out_ref.at[pl.ds(o, nr)])

            @pl.when(sid == NS - 1)
            def _():
                b = buf.at[pl.ds(0, TAILR)]
                pltpu.sync_copy(acc_ref.at[pl.ds(TAIL0, TAILR)], b)
                pltpu.sync_copy(b, out_ref.at[pl.ds(TAIL0, TAILR)])

        def process(r):
            table = tables[_REL_TABLE[r]]
            pltpu.sync_copy(zf_h, rows)
            stripe_out(rows, accf)
            pltpu.sync_copy(zc_h, cbuf)
            stripe_out(cbuf, accc)
            plsc.subcore_barrier()

            def chunk(i, carry):
                off = pl.multiple_of(ebase + i * CH, 8)
                pltpu.sync_copy(srcs[r].at[pl.ds(off, CH)], sidx)
                pltpu.sync_copy(dsts[r].at[pl.ds(off, CH)], didx)
                pltpu.async_copy(table.at[sidx], rows, sem).wait()
                pltpu.sync_copy(rows, accf.at[didx], add=True)
                pltpu.sync_copy(ones_v, accc.at[didx], add=True)
                return carry

            lax.fori_loop(0, NCHUNK, chunk, 0)
            plsc.subcore_barrier()
            stripe_flush(accf, rows, aggs[r])
            stripe_flush(accc, cbuf, cnts[r])

        for c, rels in enumerate(_CORE_RELS):
            @pl.when(cid == c)
            def _(rels=rels):
                for r in rels:
                    process(r)

    return k(*tabs, *srcs, *dsts, zf, zc, ones_h)


_BM = 2000
_BN = 512


def _mm_group(terms, bias, n_out):
    flags = tuple(cnt is not None for _, cnt, _ in terms)
    grid = (N // _BM, pl.cdiv(n_out, _BN))

    def body(*refs):
        out_ref = refs[-1]
        bias_ref = refs[-2]
        it = iter(refs[:-2])
        acc = None
        for has_cnt in flags:
            a_ref = next(it)
            a = a_ref[...]
            if has_cnt:
                cnt_ref = next(it)
                recip = 1.0 / jnp.maximum(cnt_ref[...][:, 0:1], 1.0)
                a = a * recip
            w_ref = next(it)
            p = jnp.dot(a, w_ref[...], preferred_element_type=jnp.float32)
            acc = p if acc is None else acc + p
        out_ref[...] = acc + bias_ref[...]

    in_specs = []
    args = []
    for a, cnt, w in terms:
        in_specs.append(pl.BlockSpec((_BM, D), lambda m, n: (m, 0)))
        args.append(a)
        if cnt is not None:
            in_specs.append(pl.BlockSpec((_BM, CNTW), lambda m, n: (m, 0)))
            args.append(cnt)
        in_specs.append(pl.BlockSpec((D, _BN), lambda m, n: (0, n)))
        args.append(w)
    in_specs.append(pl.BlockSpec((1, _BN), lambda m, n: (0, n)))
    args.append(bias.reshape(1, -1))

    return pl.pallas_call(
        body,
        grid=grid,
        in_specs=in_specs,
        out_specs=pl.BlockSpec((_BM, _BN), lambda m, n: (m, n)),
        out_shape=jax.ShapeDtypeStruct((N, n_out), jnp.float32),
        compiler_params=pltpu.CompilerParams(has_side_effects=True),
    )(*args)


def kernel(z_ticker, z_institution, z_mutual_fund, z_news,
           ei_holds_it, ei_holds_mt, ei_about_nt, ei_rev_holds_mt,
           ei_rev_holds_it, ei_rev_about_nt,
           Wl_hi, bl_hi, Wr_hi, Wl_hm, bl_hm, Wr_hm, Wl_an, bl_an, Wr_an,
           Wl_rhm, bl_rhm, Wr_rhm, Wl_rhi, bl_rhi, Wr_rhi,
           Wl_ran, bl_ran, Wr_ran):
    eis = (ei_holds_it, ei_holds_mt, ei_about_nt, ei_rev_holds_mt,
           ei_rev_holds_it, ei_rev_about_nt)
    srcs = tuple(ei[0] for ei in eis)
    dsts = tuple(ei[1] for ei in eis)
    zf = jnp.zeros((CH, D), jnp.float32)
    zc = jnp.zeros((CH, CNTW), jnp.float32)
    ones_h = jnp.ones((CH, CNTW), jnp.float32)

    outs = _sc_segment_sums(
        (z_ticker, z_institution, z_mutual_fund, z_news),
        srcs, dsts, zf, zc, ones_h)
    aggs, cnts = outs[:6], outs[6:]


    out_institution = _mm_group(
        [(aggs[0], cnts[0], Wl_hi), (z_institution, None, Wr_hi)],
        bl_hi, 2401)
    out_mutual_fund = _mm_group(
        [(aggs[1], cnts[1], Wl_hm), (z_mutual_fund, None, Wr_hm)],
        bl_hm, 1798)
    out_ticker = _mm_group(
        [(aggs[2], cnts[2], Wl_an), (aggs[3], cnts[3], Wl_rhm),
         (aggs[4], cnts[4], Wl_rhi),
         (z_ticker, None, Wr_an + Wr_rhm + Wr_rhi)],
        bl_an + bl_rhm + bl_rhi, 3658)
    out_news = _mm_group(
        [(aggs[5], cnts[5], Wl_ran), (z_news, None, Wr_ran)],
        bl_ran, 19340)

    return (out_ticker, out_institution, out_mutual_fund, out_news)


# speedup vs baseline: 5.0442x; 1.3792x over previous
import functools

import jax
import jax.numpy as jnp
from jax import lax
from jax.experimental import pallas as pl
from jax.experimental.pallas import tpu as pltpu
from jax.experimental.pallas import tpu_sc as plsc

N = 10000
D = 128
E = 320000
CNTW = 16
NS = 16
EDGES_PER_TILE = E // NS
CH = 80
NCHUNK = EDGES_PER_TILE // CH
STRIPE = 624
TAIL0 = STRIPE * NS
TAILR = N - TAIL0

_REL_TABLE = (0, 0, 3, 2, 1, 0)
_CORE_RELS = ((0, 1, 2), (3, 4, 5))


def _sc_segment_sums(tabs, srcs, dsts, zf, zc, ones_h):
    mesh = plsc.VectorSubcoreMesh(core_axis_name="c", subcore_axis_name="s",
                                  num_cores=2, num_subcores=NS)
    out_type = tuple(
        [jax.ShapeDtypeStruct((N, D), jnp.float32) for _ in range(6)]
        + [jax.ShapeDtypeStruct((N, CNTW), jnp.float32) for _ in range(6)]
    )

    @functools.partial(
        pl.kernel,
        mesh=mesh,
        out_type=out_type,
        compiler_params=pltpu.CompilerParams(use_tc_tiling_on_sc=False,
                                             has_side_effects=True),
        scratch_types=[
            pltpu.VMEM_SHARED((N, D), jnp.float32),
            pltpu.VMEM_SHARED((N, CNTW), jnp.float32),
            pltpu.VMEM((CH,), jnp.int32),
            pltpu.VMEM((CH,), jnp.int32),
            pltpu.VMEM((CH, D), jnp.float32),
            pltpu.VMEM((CH, CNTW), jnp.float32),
            pltpu.VMEM((CH, CNTW), jnp.float32),
            pltpu.SemaphoreType.DMA,
            pltpu.VMEM((CH,), jnp.int32),
            pltpu.VMEM((CH,), jnp.int32),
            pltpu.VMEM((CH, D), jnp.float32),
            pltpu.SemaphoreType.DMA,
        ],
    )
    def k(tab0, tab1, tab2, tab3,
          s0, s1, s2, s3, s4, s5, d0, d1, d2, d3, d4, d5,
          zf_h, zc_h, ones_hbm,
          a0, a1, a2, a3, a4, a5, c0, c1, c2, c3, c4, c5,
          accf, accc, sidx, didx, rows, ones_v, cbuf, sem,
          sidx2, didx2, rows2, sem2):
        cid = lax.axis_index("c")
        sid = lax.axis_index("s")
        tables = (tab0, tab1, tab2, tab3)
        srcs = (s0, s1, s2, s3, s4, s5)
        dsts = (d0, d1, d2, d3, d4, d5)
        aggs = (a0, a1, a2, a3, a4, a5)
        cnts = (c0, c1, c2, c3, c4, c5)
        pltpu.sync_copy(ones_hbm, ones_v)
        row0 = pl.multiple_of(sid * STRIPE, 8)
        ebase = pl.multiple_of(sid * EDGES_PER_TILE, 8)

        def stripe_chunks():
            return [(pl.multiple_of(row0 + o, 8), min(CH, STRIPE - o))
                    for o in range(0, STRIPE, CH)]

        def stripe_out(buf, dst_ref):
            for o, nr in stripe_chunks():
                pltpu.sync_copy(buf.at[pl.ds(0, nr)], dst_ref.at[pl.ds(o, nr)])

            @pl.when(sid == NS - 1)
            def _():
                pltpu.sync_copy(buf.at[pl.ds(0, TAILR)],
                                dst_ref.at[pl.ds(TAIL0, TAILR)])

        def stripe_flush(acc_ref, buf, out_ref):
            for o, nr in stripe_chunks():
                pltpu.sync_copy(acc_ref.at[pl.ds(o, nr)], buf.at[pl.ds(0, nr)])
                pltpu.sync_copy(buf.at[pl.ds(0, nr)], out_ref.at[pl.ds(o, nr)])

            @pl.when(sid == NS - 1)
            def _():
                b = buf.at[pl.ds(0, TAILR)]
                pltpu.sync_copy(acc_ref.at[pl.ds(TAIL0, TAILR)], b)
                pltpu.sync_copy(b, out_ref.at[pl.ds(TAIL0, TAILR)])

        def process(r):
            table = tables[_REL_TABLE[r]]
            pltpu.sync_copy(zf_h, rows)
            stripe_out(rows, accf)
            pltpu.sync_copy(zc_h, cbuf)
            stripe_out(cbuf, accc)
            plsc.subcore_barrier()

            def load_idx(off, sb, db):
                pltpu.sync_copy(srcs[r].at[pl.ds(off, CH)], sb)
                pltpu.sync_copy(dsts[r].at[pl.ds(off, CH)], db)

            load_idx(ebase, sidx, didx)
            pltpu.make_async_copy(table.at[sidx], rows, sem).start()

            def chunk2(j, carry):
                offb = pl.multiple_of(ebase + (2 * j + 1) * CH, 8)
                load_idx(offb, sidx2, didx2)
                pltpu.make_async_copy(table.at[sidx2], rows2, sem2).start()
                pltpu.make_async_copy(table.at[sidx], rows, sem).wait()
                pltpu.sync_copy(rows, accf.at[didx], add=True)
                pltpu.sync_copy(ones_v, accc.at[didx], add=True)

                @pl.when(j < NCHUNK // 2 - 1)
                def _():
                    offa = pl.multiple_of(ebase + (2 * j + 2) * CH, 8)
                    load_idx(offa, sidx, didx)
                    pltpu.make_async_copy(table.at[sidx], rows, sem).start()

                pltpu.make_async_copy(table.at[sidx2], rows2, sem2).wait()
                pltpu.sync_copy(rows2, accf.at[didx2], add=True)
                pltpu.sync_copy(ones_v, accc.at[didx2], add=True)
                return carry

            lax.fori_loop(0, NCHUNK // 2, chunk2, 0)
            plsc.subcore_barrier()
            stripe_flush(accf, rows, aggs[r])
            stripe_flush(accc, cbuf, cnts[r])

        for c, rels in enumerate(_CORE_RELS):
            @pl.when(cid == c)
            def _(rels=rels):
                for r in rels:
                    process(r)

    return k(*tabs, *srcs, *dsts, zf, zc, ones_h)


_BM = 2000
_BN = 512


def _mm_group(terms, bias, n_out):
    flags = tuple(cnt is not None for _, cnt, _ in terms)
    grid = (N // _BM, pl.cdiv(n_out, _BN))

    def body(*refs):
        out_ref = refs[-1]
        bias_ref = refs[-2]
        it = iter(refs[:-2])
        acc = None
        for has_cnt in flags:
            a_ref = next(it)
            a = a_ref[...]
            if has_cnt:
                cnt_ref = next(it)
                recip = 1.0 / jnp.maximum(cnt_ref[...][:, 0:1], 1.0)
                a = a * recip
            w_ref = next(it)
            p = jnp.dot(a, w_ref[...], preferred_element_type=jnp.float32)
            acc = p if acc is None else acc + p
        out_ref[...] = acc + bias_ref[...]

    in_specs = []
    args = []
    for a, cnt, w in terms:
        in_specs.append(pl.BlockSpec((_BM, D), lambda m, n: (m, 0)))
        args.append(a)
        if cnt is not None:
            in_specs.append(pl.BlockSpec((_BM, CNTW), lambda m, n: (m, 0)))
            args.append(cnt)
        in_specs.append(pl.BlockSpec((D, _BN), lambda m, n: (0, n)))
        args.append(w)
    in_specs.append(pl.BlockSpec((1, _BN), lambda m, n: (0, n)))
    args.append(bias.reshape(1, -1))

    return pl.pallas_call(
        body,
        grid=grid,
        in_specs=in_specs,
        out_specs=pl.BlockSpec((_BM, _BN), lambda m, n: (m, n)),
        out_shape=jax.ShapeDtypeStruct((N, n_out), jnp.float32),
        compiler_params=pltpu.CompilerParams(has_side_effects=True),
    )(*args)


def kernel(z_ticker, z_institution, z_mutual_fund, z_news,
           ei_holds_it, ei_holds_mt, ei_about_nt, ei_rev_holds_mt,
           ei_rev_holds_it, ei_rev_about_nt,
           Wl_hi, bl_hi, Wr_hi, Wl_hm, bl_hm, Wr_hm, Wl_an, bl_an, Wr_an,
           Wl_rhm, bl_rhm, Wr_rhm, Wl_rhi, bl_rhi, Wr_rhi,
           Wl_ran, bl_ran, Wr_ran):
    eis = (ei_holds_it, ei_holds_mt, ei_about_nt, ei_rev_holds_mt,
           ei_rev_holds_it, ei_rev_about_nt)
    srcs = tuple(ei[0] for ei in eis)
    dsts = tuple(ei[1] for ei in eis)
    zf = jnp.zeros((CH, D), jnp.float32)
    zc = jnp.zeros((CH, CNTW), jnp.float32)
    ones_h = jnp.ones((CH, CNTW), jnp.float32)

    outs = _sc_segment_sums(
        (z_ticker, z_institution, z_mutual_fund, z_news),
        srcs, dsts, zf, zc, ones_h)
    aggs, cnts = outs[:6], outs[6:]


    out_institution = _mm_group(
        [(aggs[0], cnts[0], Wl_hi), (z_institution, None, Wr_hi)],
        bl_hi, 2401)
    out_mutual_fund = _mm_group(
        [(aggs[1], cnts[1], Wl_hm), (z_mutual_fund, None, Wr_hm)],
        bl_hm, 1798)
    out_ticker = _mm_group(
        [(aggs[2], cnts[2], Wl_an), (aggs[3], cnts[3], Wl_rhm),
         (aggs[4], cnts[4], Wl_rhi),
         (z_ticker, None, Wr_an + Wr_rhm + Wr_rhi)],
        bl_an + bl_rhm + bl_rhi, 3658)
    out_news = _mm_group(
        [(aggs[5], cnts[5], Wl_ran), (z_news, None, Wr_ran)],
        bl_ran, 19340)

    return (out_ticker, out_institution, out_mutual_fund, out_news)
